# baseline (speedup 1.0000x reference)
import functools

import jax
import jax.numpy as jnp
import numpy as np
from jax.experimental import pallas as pl
from jax.experimental.pallas import tpu as pltpu

NPG = 4096
D = 128
H = 4
DH = D // H
B = 4
N = B * NPG
E = N * 32
BN = 512
F32 = jnp.float32


def _gelu(x):
    return 0.5 * x * (1.0 + jax.lax.erf(x * np.float32(1.0 / np.sqrt(2.0))))


def _l1_body(h0, a1, W1, b1, out):
    z = jnp.zeros((BN, 256), F32)
    for p in range(2):
        z = z + jax.lax.dot_general(h0[p], W1[p * 64:(p + 1) * 64, :],
                                    (((1,), (0,)), ((), ())),
                                    preferred_element_type=F32)
    for q in range(4):
        x = a1[0, q] + a1[1, q]
        z = z + jax.lax.dot_general(x, W1[q * 32:(q + 1) * 32, :],
                                    (((1,), (0,)), ((), ())),
                                    preferred_element_type=F32)
    hv = _gelu(z + b1[0][None, :])
    for p in range(4):
        out[p] = hv[:, p * 64:(p + 1) * 64]


def _run_l1(h0, a1, W1, b1):
    return pl.pallas_call(
        _l1_body,
        grid=(N // BN,),
        in_specs=[
            pl.BlockSpec((2, BN, 64), lambda i: (0, i, 0)),
            pl.BlockSpec((2, 4, BN, 32), lambda i: (0, 0, i, 0)),
            pl.BlockSpec((128, 256), lambda i: (0, 0)),
            pl.BlockSpec((1, 256), lambda i: (0, 0)),
        ],
        out_specs=pl.BlockSpec((4, BN, 64), lambda i: (0, i, 0)),
        out_shape=jax.ShapeDtypeStruct((4, N, 64), F32),
    )(h0, a1, W1, b1)


def _l2_body(h1, a2, W2, b2, W3, out):
    z = jnp.zeros((BN, 256), F32)
    for p in range(4):
        z = z + jax.lax.dot_general(h1[p], W2[p * 64:(p + 1) * 64, :],
                                    (((1,), (0,)), ((), ())),
                                    preferred_element_type=F32)
    for q in range(8):
        x = a2[0, q] + a2[1, q]
        z = z + jax.lax.dot_general(x, W2[q * 32:(q + 1) * 32, :],
                                    (((1,), (0,)), ((), ())),
                                    preferred_element_type=F32)
    h2 = _gelu(z + b2[0][None, :])
    z3 = jax.lax.dot_general(h2, W3[...], (((1,), (0,)), ((), ())),
                             preferred_element_type=F32)
    for p in range(2):
        out[p] = z3[:, p * 64:(p + 1) * 64]


def _run_l2(h1, a2, W2, b2, W3):
    return pl.pallas_call(
        _l2_body,
        grid=(N // BN,),
        in_specs=[
            pl.BlockSpec((4, BN, 64), lambda i: (0, i, 0)),
            pl.BlockSpec((2, 8, BN, 32), lambda i: (0, 0, i, 0)),
            pl.BlockSpec((256, 256), lambda i: (0, 0)),
            pl.BlockSpec((1, 256), lambda i: (0, 0)),
            pl.BlockSpec((256, 128), lambda i: (0, 0)),
        ],
        out_specs=pl.BlockSpec((2, BN, 64), lambda i: (0, i, 0)),
        out_shape=jax.ShapeDtypeStruct((2, N, 64), F32),
    )(h1, a2, W2, b2, W3)


def _l3_body(z3, a3, b3, Rm, h3_out, s_out):
    s = jnp.zeros((4, BN), F32)
    for p in range(2):
        ag = jnp.concatenate(
            [a3[0, 2 * p] + a3[1, 2 * p], a3[0, 2 * p + 1] + a3[1, 2 * p + 1]],
            axis=-1)
        hp = _gelu(z3[p] + ag + b3[0][None, p * 64:(p + 1) * 64])
        h3_out[p] = hp
        s = s + jax.lax.dot_general(Rm[p * 64:(p + 1) * 64, :], hp,
                                    (((0,), (1,)), ((), ())),
                                    preferred_element_type=F32)
    s_out[...] = s


def _run_l3(z3, a3, b3, Rm):
    return pl.pallas_call(
        _l3_body,
        grid=(N // BN,),
        in_specs=[
            pl.BlockSpec((2, BN, 64), lambda i: (0, i, 0)),
            pl.BlockSpec((2, 4, BN, 32), lambda i: (0, 0, i, 0)),
            pl.BlockSpec((1, 128), lambda i: (0, 0)),
            pl.BlockSpec((128, 4), lambda i: (0, 0)),
        ],
        out_specs=[
            pl.BlockSpec((2, BN, 64), lambda i: (0, i, 0)),
            pl.BlockSpec((4, BN), lambda i: (0, i)),
        ],
        out_shape=[
            jax.ShapeDtypeStruct((2, N, 64), F32),
            jax.ShapeDtypeStruct((4, N), F32),
        ],
    )(z3, a3, b3, Rm)


def _pre_body(cls, Wqkv, bqkv, Wout, bout, bert, Wbp, bbp, lbg, lbb,
              Rm_o, scls_o, btok_o, sbert_o, Mflat_o, c_o):
    q = jax.lax.dot_general(cls[...], Wqkv[:, 0:D], (((1,), (0,)), ((), ())),
                            preferred_element_type=F32) + bqkv[0][None, 0:D]
    Wk = Wqkv[:, D:2 * D]
    Wv = Wqkv[:, 2 * D:3 * D]
    cols = []
    for h in range(H):
        qh = q[:, h * DH:(h + 1) * DH]
        col = jax.lax.dot_general(Wk[:, h * DH:(h + 1) * DH], qh,
                                  (((1,), (1,)), ((), ())),
                                  preferred_element_type=F32)
        cols.append(col)
    Rm = jnp.concatenate(cols, axis=1) * (1.0 / np.sqrt(DH))
    Rm_o[...] = Rm
    scls_o[...] = jax.lax.dot_general(cls[...], Rm, (((1,), (0,)), ((), ())),
                                      preferred_element_type=F32)
    bt = jax.lax.dot_general(bert[...], Wbp[...], (((1,), (0,)), ((), ())),
                             preferred_element_type=F32) + bbp[0][None, :]
    mu = jnp.mean(bt, axis=-1, keepdims=True)
    var = jnp.mean((bt - mu) ** 2, axis=-1, keepdims=True)
    bt = (bt - mu) / jnp.sqrt(var + 1e-5) * lbg[0][None, :] + lbb[0][None, :]
    btok_o[...] = bt
    sbert_o[...] = jax.lax.dot_general(bt, Rm, (((1,), (0,)), ((), ())),
                                       preferred_element_type=F32)
    rows = []
    for h in range(H):
        rows.append(jax.lax.dot_general(Wv[:, h * DH:(h + 1) * DH],
                                        Wout[h * DH:(h + 1) * DH, :],
                                        (((1,), (0,)), ((), ())),
                                        preferred_element_type=F32))
    Mflat_o[...] = jnp.concatenate(rows, axis=0)
    c_o[...] = jax.lax.dot_general(bqkv[:, 2 * D:3 * D], Wout[...],
                                   (((1,), (0,)), ((), ())),
                                   preferred_element_type=F32) + bout[...]


def _run_pre(cls, Wqkv, bqkv, Wout, bout, bert, Wbp, bbp, lbg, lbb):
    full = lambda s: pl.BlockSpec(s, lambda: tuple(0 for _ in s))
    return pl.pallas_call(
        _pre_body,
        grid=(),
        in_specs=[full((1, 128)), full((128, 384)), full((1, 384)),
                  full((128, 128)), full((1, 128)), full((4, 1536)),
                  full((1536, 128)), full((1, 128)), full((1, 128)), full((1, 128))],
        out_specs=[full((128, 4)), full((1, 4)), full((4, 128)), full((4, 4)),
                   full((512, 128)), full((1, 128))],
        out_shape=[
            jax.ShapeDtypeStruct((128, 4), F32),
            jax.ShapeDtypeStruct((1, 4), F32),
            jax.ShapeDtypeStruct((4, 128), F32),
            jax.ShapeDtypeStruct((4, 4), F32),
            jax.ShapeDtypeStruct((512, 128), F32),
            jax.ShapeDtypeStruct((1, 128), F32),
        ],
    )(cls, Wqkv, bqkv, Wout, bout, bert, Wbp, bbp, lbg, lbb)


def _stats_body(s, scls, sbert, m_o, z_o):
    mx = jnp.max(s[...], axis=1)
    m = jnp.maximum(mx, jnp.maximum(scls[0], sbert[0, 0]))
    zs = jnp.sum(jnp.exp(s[...] - m[:, None]), axis=1)
    z = zs + jnp.exp(scls[0] - m) + jnp.exp(sbert[0, 0] - m)
    m_o[...] = m[None, None, :]
    z_o[...] = z[None, None, :]


def _run_stats(s, scls, sbert):
    return pl.pallas_call(
        _stats_body,
        grid=(B,),
        in_specs=[
            pl.BlockSpec((4, NPG), lambda b: (0, b)),
            pl.BlockSpec((1, 4), lambda b: (0, 0)),
            pl.BlockSpec((1, 1, 4), lambda b: (b, 0, 0)),
        ],
        out_specs=[pl.BlockSpec((1, 1, 4), lambda b: (b, 0, 0)),
                   pl.BlockSpec((1, 1, 4), lambda b: (b, 0, 0))],
        out_shape=[jax.ShapeDtypeStruct((B, 1, 4), F32),
                   jax.ShapeDtypeStruct((B, 1, 4), F32)],
    )(s, scls, sbert)


BN2 = 2048
NC2 = NPG // BN2


def _u_body(s, h3, m, z, scls, sbert, cls, btok, u_o):
    j = pl.program_id(1)
    a = jnp.exp(s[...] - m[0, 0][:, None]) / z[0, 0][:, None]
    parts = []
    for p in range(2):
        parts.append(jax.lax.dot_general(a, h3[p], (((1,), (0,)), ((), ())),
                                         preferred_element_type=F32))
    u = jnp.concatenate(parts, axis=1)

    @pl.when(j == 0)
    def _init():
        ec = jnp.exp(scls[0] - m[0, 0]) / z[0, 0]
        eb = jnp.exp(sbert[0, 0] - m[0, 0]) / z[0, 0]
        u_o[0] = (u + ec[:, None] * cls[0][None, :]
                  + eb[:, None] * btok[0, 0][None, :])

    @pl.when(j != 0)
    def _acc():
        u_o[0] = u_o[0] + u


def _run_u(s, h3, m, z, scls, sbert, cls, btok):
    return pl.pallas_call(
        _u_body,
        grid=(B, NC2),
        in_specs=[
            pl.BlockSpec((4, BN2), lambda b, j: (0, b * NC2 + j)),
            pl.BlockSpec((2, BN2, 64), lambda b, j: (0, b * NC2 + j, 0)),
            pl.BlockSpec((1, 1, 4), lambda b, j: (b, 0, 0)),
            pl.BlockSpec((1, 1, 4), lambda b, j: (b, 0, 0)),
            pl.BlockSpec((1, 4), lambda b, j: (0, 0)),
            pl.BlockSpec((1, 1, 4), lambda b, j: (b, 0, 0)),
            pl.BlockSpec((1, 128), lambda b, j: (0, 0)),
            pl.BlockSpec((1, 1, 128), lambda b, j: (b, 0, 0)),
        ],
        out_specs=pl.BlockSpec((1, 4, 128), lambda b, j: (b, 0, 0)),
        out_shape=jax.ShapeDtypeStruct((B, 4, 128), F32),
    )(s, h3, m, z, scls, sbert, cls, btok)


def _fin_body(u, Mflat, c, cls, lag, lab, out):
    x = c[0][None, :] + cls[...]
    acc = jnp.zeros((B, D), F32)
    for h in range(H):
        acc = acc + jax.lax.dot_general(u[:, h, :], Mflat[h * D:(h + 1) * D, :],
                                        (((1,), (0,)), ((), ())),
                                        preferred_element_type=F32)
    y = acc + x
    mu = jnp.mean(y, axis=-1, keepdims=True)
    var = jnp.mean((y - mu) ** 2, axis=-1, keepdims=True)
    out[...] = (y - mu) / jnp.sqrt(var + 1e-5) * lag[0][None, :] + lab[0][None, :]


def _run_fin(u, Mflat, c, cls, lag, lab):
    full = lambda s: pl.BlockSpec(s, lambda: tuple(0 for _ in s))
    return pl.pallas_call(
        _fin_body,
        grid=(),
        in_specs=[full((B, 4, 128)), full((512, 128)), full((1, 128)),
                  full((1, 128)), full((1, 128)), full((1, 128))],
        out_specs=full((B, 128)),
        out_shape=jax.ShapeDtypeStruct((B, 128), F32),
    )(u, Mflat, c, cls, lag, lab)


NW = 32
NPW = N // NW
EPT = E // NW
CH = 128
NCH = EPT // CH
RPT = N // 16


def _sc_mesh():
    from jax.experimental.pallas import tpu_sc as plsc
    return plsc.VectorSubcoreMesh(core_axis_name="c", subcore_axis_name="s")


def _embed_body(et_hbm, mask_hbm, h0_hbm, idx_v, idx2_v, rows_v, sem):
    from jax import lax
    wid = lax.axis_index("s") * 2 + lax.axis_index("c")
    base = wid * NPW
    pltpu.sync_copy(mask_hbm.at[pl.ds(base, NPW)], idx_v)
    pltpu.async_copy(et_hbm.at[idx_v], rows_v, sem).wait()
    pltpu.sync_copy(rows_v, h0_hbm.at[0, pl.ds(base, NPW)])

    def add_off(i, _):
        idx2_v[pl.ds(i * 16, 16)] = idx_v[pl.ds(i * 16, 16)] + 4096
        return 0
    jax.lax.fori_loop(0, NPW // 16, add_off, 0)
    pltpu.async_copy(et_hbm.at[idx2_v], rows_v, sem).wait()
    pltpu.sync_copy(rows_v, h0_hbm.at[1, pl.ds(base, NPW)])


def _embed_gather(embed_table, mask):
    etflat = embed_table.reshape(4096, 2, 64).transpose(1, 0, 2).reshape(2 * 4096, 64)
    run = pl.kernel(
        _embed_body,
        out_type=jax.ShapeDtypeStruct((2, N, 64), F32),
        mesh=_sc_mesh(),
        compiler_params=pltpu.CompilerParams(use_tc_tiling_on_sc=False, needs_layout_passes=False),
        scratch_types=[
            pltpu.VMEM((NPW,), jnp.int32),
            pltpu.VMEM((NPW,), jnp.int32),
            pltpu.VMEM((NPW, 64), F32),
            pltpu.SemaphoreType.DMA,
        ],
    )
    return run(etflat, mask)


def _make_edge_body(P):
    from jax.experimental.pallas import tpu_sc as plsc
    from jax import lax

    def body(yflat_hbm, src_hbm, dst_hbm, w_hbm, out_hbm,
             src_v, dst_v, w_v, idx_v, didx_v, gb, sb, zbuf, acc,
             gsem, ssem):
        cid = lax.axis_index("c")
        sid = lax.axis_index("s")
        wid = sid * 2 + cid
        ebase = wid * EPT
        pltpu.sync_copy(src_hbm.at[pl.ds(ebase, EPT)], src_v)
        pltpu.sync_copy(dst_hbm.at[pl.ds(ebase, EPT)], dst_v)
        pltpu.sync_copy(w_hbm.at[pl.ds(ebase, EPT)], w_v)

        def zb(i, _):
            for k in range(2):
                zbuf[i, pl.ds(k * 16, 16)] = jnp.zeros((16,), F32)
            return 0
        lax.fori_loop(0, 256, zb, 0)

        NB = 4
        NG = NCH // NB

        def build_idx(b, j, off):
            for k2 in range(CH // 16):
                idx_v[b, pl.ds(k2 * 16, 16)] = (
                    src_v[pl.ds(j * CH + k2 * 16, 16)] * 2 + off)

        def build_didx(b, j):
            for k2 in range(CH // 16):
                didx_v[b, pl.ds(k2 * 16, 16)] = dst_v[pl.ds(j * CH + k2 * 16, 16)]

        def g_issue(b):
            pltpu.async_copy(yflat_hbm.at[idx_v.at[b]], gb.at[b], gsem.at[b])

        def g_wait(b):
            pltpu.make_async_copy(yflat_hbm.at[idx_v.at[b]], gb.at[b],
                                  gsem.at[b]).wait()

        def s_issue(b):
            pltpu.async_copy(sb.at[b], acc.at[didx_v.at[b]], ssem.at[b],
                             add=True)

        def s_wait(b):
            pltpu.make_async_copy(sb.at[b], acc.at[didx_v.at[b]],
                                  ssem.at[b]).wait()

        for q in range(2 * P):
            off = (q // 2) * 2 * N + (q % 2)
            for k in range(4):
                pltpu.sync_copy(zbuf, acc.at[pl.ds(sid * RPT + k * 256, 256)])
            plsc.subcore_barrier()

            for b in range(NB):
                build_idx(b, b, off)
                g_issue(b)

            def group(jo, _):
                for b in range(NB):
                    j = jo * NB + b

                    @pl.when(jo > 0)
                    def _():
                        s_wait(b)
                    g_wait(b)

                    def scale(r4, _):
                        for u in range(4):
                            r = r4 * 4 + u
                            wsp = plsc.load_gather(
                                w_v, [jnp.full((16,), j * CH + r, jnp.int32)])
                            for k in range(2):
                                sb[b, r, pl.ds(k * 16, 16)] = (
                                    gb[b, r, pl.ds(k * 16, 16)] * wsp)
                        return 0
                    lax.fori_loop(0, CH // 4, scale, 0)
                    build_didx(b, j)
                    s_issue(b)

                    @pl.when(jo < NG - 1)
                    def _():
                        build_idx(b, j + NB, off)
                        g_issue(b)
                return 0
            lax.fori_loop(0, NG, group, 0)
            for b in range(NB):
                s_wait(b)
            plsc.subcore_barrier()
            pltpu.sync_copy(acc.at[pl.ds(sid * RPT, RPT)],
                            out_hbm.at[cid, q, pl.ds(sid * RPT, RPT)])
            plsc.subcore_barrier()
    return body


def _agg_chunks(y, src, dst, w, P):
    yflat = y.reshape(2 * P * N, 32)
    run = pl.kernel(
        _make_edge_body(P),
        out_type=jax.ShapeDtypeStruct((2, 2 * P, N, 32), F32),
        mesh=_sc_mesh(),
        compiler_params=pltpu.CompilerParams(use_tc_tiling_on_sc=False, needs_layout_passes=False),
        scratch_types=[
            pltpu.VMEM((EPT,), jnp.int32),
            pltpu.VMEM((EPT,), jnp.int32),
            pltpu.VMEM((EPT,), F32),
            pltpu.VMEM((4, CH), jnp.int32),
            pltpu.VMEM((4, CH), jnp.int32),
            pltpu.VMEM((4, CH, 32), F32),
            pltpu.VMEM((4, CH, 32), F32),
            pltpu.VMEM((256, 32), F32),
            pltpu.VMEM_SHARED((N, 32), F32),
            pltpu.SemaphoreType.DMA((4,)),
            pltpu.SemaphoreType.DMA((4,)),
        ],
    )
    return run(yflat, src, dst, w)


def kernel(mask, edge_index, sw, edge_weight, bert_feat, embed_table, W1, b1,
           W2, b2, W3, b3, cls_emb, Wbp, bbp, ln_b_g, ln_b_b, Wqkv, bqkv,
           Wout, bout, ln_a_g, ln_a_b):
    src = edge_index[0]
    dst = edge_index[1]
    row = lambda v: v.reshape(1, -1)

    cls = cls_emb.reshape(1, D)
    Rm, scls, btok, sbert, Mflat, c = _run_pre(
        cls, Wqkv, row(bqkv), Wout, row(bout), bert_feat, Wbp, row(bbp),
        row(ln_b_g), row(ln_b_b))

    h0 = _embed_gather(embed_table, mask)
    a1 = _agg_chunks(h0, src, dst, edge_weight, 2)
    h1 = _run_l1(h0, a1, W1, row(b1))
    a2 = _agg_chunks(h1, src, dst, edge_weight, 4)
    z3 = _run_l2(h1, a2, W2, row(b2), W3)
    a3 = _agg_chunks(z3, src, dst, edge_weight, 2)
    h3, s = _run_l3(z3, a3, row(b3), Rm)

    sbert_r = sbert.reshape(B, 1, 4)
    btok_r = btok.reshape(B, 1, 128)
    m, z = _run_stats(s, scls, sbert_r)
    u = _run_u(s, h3, m, z, scls, sbert_r, cls, btok_r)
    return _run_fin(u, Mflat, c, cls, row(ln_a_g), row(ln_a_b))


# speedup vs baseline: 1.4044x; 1.4044x over previous
import functools

import jax
import jax.numpy as jnp
import numpy as np
from jax.experimental import pallas as pl
from jax.experimental.pallas import tpu as pltpu

NPG = 4096
D = 128
H = 4
DH = D // H
B = 4
N = B * NPG
E = N * 32
BN = 512
F32 = jnp.float32


def _gelu(x):
    return 0.5 * x * (1.0 + jax.lax.erf(x * np.float32(1.0 / np.sqrt(2.0))))


def _l1_body(h0, a1, W1, b1, out):
    z = jnp.zeros((BN, 256), F32)
    for p in range(2):
        z = z + jax.lax.dot_general(h0[p], W1[p * 64:(p + 1) * 64, :],
                                    (((1,), (0,)), ((), ())),
                                    preferred_element_type=F32)
    for q in range(4):
        x = a1[0, q] + a1[1, q]
        z = z + jax.lax.dot_general(x, W1[q * 32:(q + 1) * 32, :],
                                    (((1,), (0,)), ((), ())),
                                    preferred_element_type=F32)
    hv = _gelu(z + b1[0][None, :])
    for p in range(4):
        out[p] = hv[:, p * 64:(p + 1) * 64]


def _run_l1(h0, a1, W1, b1):
    return pl.pallas_call(
        _l1_body,
        grid=(N // BN,),
        in_specs=[
            pl.BlockSpec((2, BN, 64), lambda i: (0, i, 0)),
            pl.BlockSpec((2, 4, BN, 32), lambda i: (0, 0, i, 0)),
            pl.BlockSpec((128, 256), lambda i: (0, 0)),
            pl.BlockSpec((1, 256), lambda i: (0, 0)),
        ],
        out_specs=pl.BlockSpec((4, BN, 64), lambda i: (0, i, 0)),
        out_shape=jax.ShapeDtypeStruct((4, N, 64), F32),
    )(h0, a1, W1, b1)


def _l2_body(h1, a2, W2, b2, W3, out):
    z = jnp.zeros((BN, 256), F32)
    for p in range(4):
        z = z + jax.lax.dot_general(h1[p], W2[p * 64:(p + 1) * 64, :],
                                    (((1,), (0,)), ((), ())),
                                    preferred_element_type=F32)
    for q in range(8):
        x = a2[0, q] + a2[1, q]
        z = z + jax.lax.dot_general(x, W2[q * 32:(q + 1) * 32, :],
                                    (((1,), (0,)), ((), ())),
                                    preferred_element_type=F32)
    h2 = _gelu(z + b2[0][None, :])
    z3 = jax.lax.dot_general(h2, W3[...], (((1,), (0,)), ((), ())),
                             preferred_element_type=F32)
    for p in range(2):
        out[p] = z3[:, p * 64:(p + 1) * 64]


def _run_l2(h1, a2, W2, b2, W3):
    return pl.pallas_call(
        _l2_body,
        grid=(N // BN,),
        in_specs=[
            pl.BlockSpec((4, BN, 64), lambda i: (0, i, 0)),
            pl.BlockSpec((2, 8, BN, 32), lambda i: (0, 0, i, 0)),
            pl.BlockSpec((256, 256), lambda i: (0, 0)),
            pl.BlockSpec((1, 256), lambda i: (0, 0)),
            pl.BlockSpec((256, 128), lambda i: (0, 0)),
        ],
        out_specs=pl.BlockSpec((2, BN, 64), lambda i: (0, i, 0)),
        out_shape=jax.ShapeDtypeStruct((2, N, 64), F32),
    )(h1, a2, W2, b2, W3)


def _l3_body(z3, a3, b3, Rm, h3_out, s_out):
    s = jnp.zeros((4, BN), F32)
    for p in range(2):
        ag = jnp.concatenate(
            [a3[0, 2 * p] + a3[1, 2 * p], a3[0, 2 * p + 1] + a3[1, 2 * p + 1]],
            axis=-1)
        hp = _gelu(z3[p] + ag + b3[0][None, p * 64:(p + 1) * 64])
        h3_out[p] = hp
        s = s + jax.lax.dot_general(Rm[p * 64:(p + 1) * 64, :], hp,
                                    (((0,), (1,)), ((), ())),
                                    preferred_element_type=F32)
    s_out[...] = s


def _run_l3(z3, a3, b3, Rm):
    return pl.pallas_call(
        _l3_body,
        grid=(N // BN,),
        in_specs=[
            pl.BlockSpec((2, BN, 64), lambda i: (0, i, 0)),
            pl.BlockSpec((2, 4, BN, 32), lambda i: (0, 0, i, 0)),
            pl.BlockSpec((1, 128), lambda i: (0, 0)),
            pl.BlockSpec((128, 4), lambda i: (0, 0)),
        ],
        out_specs=[
            pl.BlockSpec((2, BN, 64), lambda i: (0, i, 0)),
            pl.BlockSpec((4, BN), lambda i: (0, i)),
        ],
        out_shape=[
            jax.ShapeDtypeStruct((2, N, 64), F32),
            jax.ShapeDtypeStruct((4, N), F32),
        ],
    )(z3, a3, b3, Rm)


def _pre_body(cls, Wqkv, bqkv, Wout, bout, bert, Wbp, bbp, lbg, lbb,
              Rm_o, scls_o, btok_o, sbert_o, Mflat_o, c_o):
    q = jax.lax.dot_general(cls[...], Wqkv[:, 0:D], (((1,), (0,)), ((), ())),
                            preferred_element_type=F32) + bqkv[0][None, 0:D]
    Wk = Wqkv[:, D:2 * D]
    Wv = Wqkv[:, 2 * D:3 * D]
    cols = []
    for h in range(H):
        qh = q[:, h * DH:(h + 1) * DH]
        col = jax.lax.dot_general(Wk[:, h * DH:(h + 1) * DH], qh,
                                  (((1,), (1,)), ((), ())),
                                  preferred_element_type=F32)
        cols.append(col)
    Rm = jnp.concatenate(cols, axis=1) * (1.0 / np.sqrt(DH))
    Rm_o[...] = Rm
    scls_o[...] = jax.lax.dot_general(cls[...], Rm, (((1,), (0,)), ((), ())),
                                      preferred_element_type=F32)
    bt = jax.lax.dot_general(bert[...], Wbp[...], (((1,), (0,)), ((), ())),
                             preferred_element_type=F32) + bbp[0][None, :]
    mu = jnp.mean(bt, axis=-1, keepdims=True)
    var = jnp.mean((bt - mu) ** 2, axis=-1, keepdims=True)
    bt = (bt - mu) / jnp.sqrt(var + 1e-5) * lbg[0][None, :] + lbb[0][None, :]
    btok_o[...] = bt
    sbert_o[...] = jax.lax.dot_general(bt, Rm, (((1,), (0,)), ((), ())),
                                       preferred_element_type=F32)
    rows = []
    for h in range(H):
        rows.append(jax.lax.dot_general(Wv[:, h * DH:(h + 1) * DH],
                                        Wout[h * DH:(h + 1) * DH, :],
                                        (((1,), (0,)), ((), ())),
                                        preferred_element_type=F32))
    Mflat_o[...] = jnp.concatenate(rows, axis=0)
    c_o[...] = jax.lax.dot_general(bqkv[:, 2 * D:3 * D], Wout[...],
                                   (((1,), (0,)), ((), ())),
                                   preferred_element_type=F32) + bout[...]


def _run_pre(cls, Wqkv, bqkv, Wout, bout, bert, Wbp, bbp, lbg, lbb):
    full = lambda s: pl.BlockSpec(s, lambda: tuple(0 for _ in s))
    return pl.pallas_call(
        _pre_body,
        grid=(),
        in_specs=[full((1, 128)), full((128, 384)), full((1, 384)),
                  full((128, 128)), full((1, 128)), full((4, 1536)),
                  full((1536, 128)), full((1, 128)), full((1, 128)), full((1, 128))],
        out_specs=[full((128, 4)), full((1, 4)), full((4, 128)), full((4, 4)),
                   full((512, 128)), full((1, 128))],
        out_shape=[
            jax.ShapeDtypeStruct((128, 4), F32),
            jax.ShapeDtypeStruct((1, 4), F32),
            jax.ShapeDtypeStruct((4, 128), F32),
            jax.ShapeDtypeStruct((4, 4), F32),
            jax.ShapeDtypeStruct((512, 128), F32),
            jax.ShapeDtypeStruct((1, 128), F32),
        ],
    )(cls, Wqkv, bqkv, Wout, bout, bert, Wbp, bbp, lbg, lbb)


def _stats_body(s, scls, sbert, m_o, z_o):
    mx = jnp.max(s[...], axis=1)
    m = jnp.maximum(mx, jnp.maximum(scls[0], sbert[0, 0]))
    zs = jnp.sum(jnp.exp(s[...] - m[:, None]), axis=1)
    z = zs + jnp.exp(scls[0] - m) + jnp.exp(sbert[0, 0] - m)
    m_o[...] = m[None, None, :]
    z_o[...] = z[None, None, :]


def _run_stats(s, scls, sbert):
    return pl.pallas_call(
        _stats_body,
        grid=(B,),
        in_specs=[
            pl.BlockSpec((4, NPG), lambda b: (0, b)),
            pl.BlockSpec((1, 4), lambda b: (0, 0)),
            pl.BlockSpec((1, 1, 4), lambda b: (b, 0, 0)),
        ],
        out_specs=[pl.BlockSpec((1, 1, 4), lambda b: (b, 0, 0)),
                   pl.BlockSpec((1, 1, 4), lambda b: (b, 0, 0))],
        out_shape=[jax.ShapeDtypeStruct((B, 1, 4), F32),
                   jax.ShapeDtypeStruct((B, 1, 4), F32)],
    )(s, scls, sbert)


BN2 = 2048
NC2 = NPG // BN2


def _u_body(s, h3, m, z, scls, sbert, cls, btok, u_o):
    j = pl.program_id(1)
    a = jnp.exp(s[...] - m[0, 0][:, None]) / z[0, 0][:, None]
    parts = []
    for p in range(2):
        parts.append(jax.lax.dot_general(a, h3[p], (((1,), (0,)), ((), ())),
                                         preferred_element_type=F32))
    u = jnp.concatenate(parts, axis=1)

    @pl.when(j == 0)
    def _init():
        ec = jnp.exp(scls[0] - m[0, 0]) / z[0, 0]
        eb = jnp.exp(sbert[0, 0] - m[0, 0]) / z[0, 0]
        u_o[0] = (u + ec[:, None] * cls[0][None, :]
                  + eb[:, None] * btok[0, 0][None, :])

    @pl.when(j != 0)
    def _acc():
        u_o[0] = u_o[0] + u


def _run_u(s, h3, m, z, scls, sbert, cls, btok):
    return pl.pallas_call(
        _u_body,
        grid=(B, NC2),
        in_specs=[
            pl.BlockSpec((4, BN2), lambda b, j: (0, b * NC2 + j)),
            pl.BlockSpec((2, BN2, 64), lambda b, j: (0, b * NC2 + j, 0)),
            pl.BlockSpec((1, 1, 4), lambda b, j: (b, 0, 0)),
            pl.BlockSpec((1, 1, 4), lambda b, j: (b, 0, 0)),
            pl.BlockSpec((1, 4), lambda b, j: (0, 0)),
            pl.BlockSpec((1, 1, 4), lambda b, j: (b, 0, 0)),
            pl.BlockSpec((1, 128), lambda b, j: (0, 0)),
            pl.BlockSpec((1, 1, 128), lambda b, j: (b, 0, 0)),
        ],
        out_specs=pl.BlockSpec((1, 4, 128), lambda b, j: (b, 0, 0)),
        out_shape=jax.ShapeDtypeStruct((B, 4, 128), F32),
    )(s, h3, m, z, scls, sbert, cls, btok)


def _fin_body(u, Mflat, c, cls, lag, lab, out):
    x = c[0][None, :] + cls[...]
    acc = jnp.zeros((B, D), F32)
    for h in range(H):
        acc = acc + jax.lax.dot_general(u[:, h, :], Mflat[h * D:(h + 1) * D, :],
                                        (((1,), (0,)), ((), ())),
                                        preferred_element_type=F32)
    y = acc + x
    mu = jnp.mean(y, axis=-1, keepdims=True)
    var = jnp.mean((y - mu) ** 2, axis=-1, keepdims=True)
    out[...] = (y - mu) / jnp.sqrt(var + 1e-5) * lag[0][None, :] + lab[0][None, :]


def _run_fin(u, Mflat, c, cls, lag, lab):
    full = lambda s: pl.BlockSpec(s, lambda: tuple(0 for _ in s))
    return pl.pallas_call(
        _fin_body,
        grid=(),
        in_specs=[full((B, 4, 128)), full((512, 128)), full((1, 128)),
                  full((1, 128)), full((1, 128)), full((1, 128))],
        out_specs=full((B, 128)),
        out_shape=jax.ShapeDtypeStruct((B, 128), F32),
    )(u, Mflat, c, cls, lag, lab)


NW = 32
NPW = N // NW
EPT = E // NW
CH = 128
NCH = EPT // CH
RPT = N // 16


def _sc_mesh():
    from jax.experimental.pallas import tpu_sc as plsc
    return plsc.VectorSubcoreMesh(core_axis_name="c", subcore_axis_name="s")


def _embed_body(et_hbm, mask_hbm, h0_hbm, idx_v, idx2_v, rows_v, sem):
    from jax import lax
    wid = lax.axis_index("s") * 2 + lax.axis_index("c")
    base = wid * NPW
    pltpu.sync_copy(mask_hbm.at[pl.ds(base, NPW)], idx_v)
    pltpu.async_copy(et_hbm.at[idx_v], rows_v, sem).wait()
    pltpu.sync_copy(rows_v, h0_hbm.at[0, pl.ds(base, NPW)])

    def add_off(i, _):
        idx2_v[pl.ds(i * 16, 16)] = idx_v[pl.ds(i * 16, 16)] + 4096
        return 0
    jax.lax.fori_loop(0, NPW // 16, add_off, 0)
    pltpu.async_copy(et_hbm.at[idx2_v], rows_v, sem).wait()
    pltpu.sync_copy(rows_v, h0_hbm.at[1, pl.ds(base, NPW)])


def _embed_gather(embed_table, mask):
    etflat = embed_table.reshape(4096, 2, 64).transpose(1, 0, 2).reshape(2 * 4096, 64)
    run = pl.kernel(
        _embed_body,
        out_type=jax.ShapeDtypeStruct((2, N, 64), F32),
        mesh=_sc_mesh(),
        compiler_params=pltpu.CompilerParams(use_tc_tiling_on_sc=False, needs_layout_passes=False),
        scratch_types=[
            pltpu.VMEM((NPW,), jnp.int32),
            pltpu.VMEM((NPW,), jnp.int32),
            pltpu.VMEM((NPW, 64), F32),
            pltpu.SemaphoreType.DMA,
        ],
    )
    return run(etflat, mask)


def _make_edge_body(P):
    from jax.experimental.pallas import tpu_sc as plsc
    from jax import lax

    def body(yflat_hbm, src_hbm, dst_hbm, w_hbm, out_hbm,
             src_v, dst_v, w_v, idx_v, didx_v, gb, sb, zbuf, acc,
             gsem, ssem):
        cid = lax.axis_index("c")
        sid = lax.axis_index("s")
        wid = sid * 2 + cid
        ebase = wid * EPT
        pltpu.sync_copy(src_hbm.at[pl.ds(ebase, EPT)], src_v)
        pltpu.sync_copy(dst_hbm.at[pl.ds(ebase, EPT)], dst_v)
        pltpu.sync_copy(w_hbm.at[pl.ds(ebase, EPT)], w_v)

        def zb(i, _):
            for k in range(2):
                zbuf[i, pl.ds(k * 16, 16)] = jnp.zeros((16,), F32)
            return 0
        lax.fori_loop(0, 256, zb, 0)

        NB = 4
        NG = NCH // NB

        def build_idx(b, j, off):
            for k2 in range(CH // 16):
                idx_v[b, pl.ds(k2 * 16, 16)] = (
                    src_v[pl.ds(j * CH + k2 * 16, 16)] * 2 + off)

        def build_didx(b, j):
            for k2 in range(CH // 16):
                didx_v[b, pl.ds(k2 * 16, 16)] = dst_v[pl.ds(j * CH + k2 * 16, 16)]

        def g_issue(b):
            pltpu.async_copy(yflat_hbm.at[idx_v.at[b]], gb.at[b], gsem.at[b])

        def g_wait(b):
            pltpu.make_async_copy(yflat_hbm.at[idx_v.at[b]], gb.at[b],
                                  gsem.at[b]).wait()

        def s_issue(b):
            pltpu.async_copy(sb.at[b], acc.at[didx_v.at[b]], ssem.at[b],
                             add=True)

        def s_wait(b):
            pltpu.make_async_copy(sb.at[b], acc.at[didx_v.at[b]],
                                  ssem.at[b]).wait()

        for q in range(2 * P):
            off = (q // 2) * 2 * N + (q % 2)
            for k in range(4):
                pltpu.sync_copy(zbuf, acc.at[pl.ds(sid * RPT + k * 256, 256)])
            plsc.subcore_barrier()

            for b in range(NB):
                build_idx(b, b, off)
                g_issue(b)

            def group(jo, _):
                for b in range(NB):
                    j = jo * NB + b

                    @pl.when(jo > 0)
                    def _():
                        s_wait(b)
                    g_wait(b)

                    def scale(r, _):
                        wsp = plsc.load_gather(
                            w_v, [jnp.full((16,), j * CH + r, jnp.int32)])
                        for k in range(2):
                            sb[b, r, pl.ds(k * 16, 16)] = (
                                gb[b, r, pl.ds(k * 16, 16)] * wsp)
                        return 0
                    lax.fori_loop(0, CH, scale, 0)
                    build_didx(b, j)
                    s_issue(b)

                    @pl.when(jo < NG - 1)
                    def _():
                        build_idx(b, j + NB, off)
                        g_issue(b)
                return 0
            lax.fori_loop(0, NG, group, 0)
            for b in range(NB):
                s_wait(b)
            plsc.subcore_barrier()
            pltpu.sync_copy(acc.at[pl.ds(sid * RPT, RPT)],
                            out_hbm.at[cid, q, pl.ds(sid * RPT, RPT)])
            plsc.subcore_barrier()
    return body


def _agg_chunks(y, src, dst, w, P):
    yflat = y.reshape(2 * P * N, 32)
    run = pl.kernel(
        _make_edge_body(P),
        out_type=jax.ShapeDtypeStruct((2, 2 * P, N, 32), F32),
        mesh=_sc_mesh(),
        compiler_params=pltpu.CompilerParams(use_tc_tiling_on_sc=False, needs_layout_passes=False),
        scratch_types=[
            pltpu.VMEM((EPT,), jnp.int32),
            pltpu.VMEM((EPT,), jnp.int32),
            pltpu.VMEM((EPT,), F32),
            pltpu.VMEM((4, CH), jnp.int32),
            pltpu.VMEM((4, CH), jnp.int32),
            pltpu.VMEM((4, CH, 32), F32),
            pltpu.VMEM((4, CH, 32), F32),
            pltpu.VMEM((256, 32), F32),
            pltpu.VMEM_SHARED((N, 32), F32),
            pltpu.SemaphoreType.DMA((4,)),
            pltpu.SemaphoreType.DMA((4,)),
        ],
    )
    return run(yflat, src, dst, w)


def kernel(mask, edge_index, sw, edge_weight, bert_feat, embed_table, W1, b1,
           W2, b2, W3, b3, cls_emb, Wbp, bbp, ln_b_g, ln_b_b, Wqkv, bqkv,
           Wout, bout, ln_a_g, ln_a_b):
    src = edge_index[0]
    dst = edge_index[1]
    row = lambda v: v.reshape(1, -1)

    cls = cls_emb.reshape(1, D)
    Rm, scls, btok, sbert, Mflat, c = _run_pre(
        cls, Wqkv, row(bqkv), Wout, row(bout), bert_feat, Wbp, row(bbp),
        row(ln_b_g), row(ln_b_b))

    h0 = _embed_gather(embed_table, mask)
    a1 = _agg_chunks(h0, src, dst, edge_weight, 2)
    h1 = _run_l1(h0, a1, W1, row(b1))
    a2 = _agg_chunks(h1, src, dst, edge_weight, 4)
    z3 = _run_l2(h1, a2, W2, row(b2), W3)
    a3 = _agg_chunks(z3, src, dst, edge_weight, 2)
    h3, s = _run_l3(z3, a3, row(b3), Rm)

    sbert_r = sbert.reshape(B, 1, 4)
    btok_r = btok.reshape(B, 1, 128)
    m, z = _run_stats(s, scls, sbert_r)
    u = _run_u(s, h3, m, z, scls, sbert_r, cls, btok_r)
    return _run_fin(u, Mflat, c, cls, row(ln_a_g), row(ln_a_b))


# speedup vs baseline: 2.6775x; 1.9066x over previous
import functools

import jax
import jax.numpy as jnp
import numpy as np
from jax.experimental import pallas as pl
from jax.experimental.pallas import tpu as pltpu

NPG = 4096
D = 128
H = 4
DH = D // H
B = 4
N = B * NPG
E = N * 32
BN = 512
F32 = jnp.float32


def _gelu(x):
    return 0.5 * x * (1.0 + jax.lax.erf(x * np.float32(1.0 / np.sqrt(2.0))))


def _l1_body(h0, a1, W1, b1, out):
    z = jnp.zeros((BN, 256), F32)
    for p in range(2):
        z = z + jax.lax.dot_general(h0[p], W1[p * 64:(p + 1) * 64, :],
                                    (((1,), (0,)), ((), ())),
                                    preferred_element_type=F32)
    for q in range(4):
        x = a1[0, q] + a1[1, q]
        z = z + jax.lax.dot_general(x, W1[q * 32:(q + 1) * 32, :],
                                    (((1,), (0,)), ((), ())),
                                    preferred_element_type=F32)
    hv = _gelu(z + b1[0][None, :])
    for p in range(4):
        out[p] = hv[:, p * 64:(p + 1) * 64]


def _run_l1(h0, a1, W1, b1):
    return pl.pallas_call(
        _l1_body,
        grid=(N // BN,),
        in_specs=[
            pl.BlockSpec((2, BN, 64), lambda i: (0, i, 0)),
            pl.BlockSpec((2, 4, BN, 32), lambda i: (0, 0, i, 0)),
            pl.BlockSpec((128, 256), lambda i: (0, 0)),
            pl.BlockSpec((1, 256), lambda i: (0, 0)),
        ],
        out_specs=pl.BlockSpec((4, BN, 64), lambda i: (0, i, 0)),
        out_shape=jax.ShapeDtypeStruct((4, N, 64), F32),
    )(h0, a1, W1, b1)


def _l2_body(h1, a2, W2, b2, W3, out):
    z = jnp.zeros((BN, 256), F32)
    for p in range(4):
        z = z + jax.lax.dot_general(h1[p], W2[p * 64:(p + 1) * 64, :],
                                    (((1,), (0,)), ((), ())),
                                    preferred_element_type=F32)
    for q in range(8):
        x = a2[0, q] + a2[1, q]
        z = z + jax.lax.dot_general(x, W2[q * 32:(q + 1) * 32, :],
                                    (((1,), (0,)), ((), ())),
                                    preferred_element_type=F32)
    h2 = _gelu(z + b2[0][None, :])
    z3 = jax.lax.dot_general(h2, W3[...], (((1,), (0,)), ((), ())),
                             preferred_element_type=F32)
    for p in range(2):
        out[p] = z3[:, p * 64:(p + 1) * 64]


def _run_l2(h1, a2, W2, b2, W3):
    return pl.pallas_call(
        _l2_body,
        grid=(N // BN,),
        in_specs=[
            pl.BlockSpec((4, BN, 64), lambda i: (0, i, 0)),
            pl.BlockSpec((2, 8, BN, 32), lambda i: (0, 0, i, 0)),
            pl.BlockSpec((256, 256), lambda i: (0, 0)),
            pl.BlockSpec((1, 256), lambda i: (0, 0)),
            pl.BlockSpec((256, 128), lambda i: (0, 0)),
        ],
        out_specs=pl.BlockSpec((2, BN, 64), lambda i: (0, i, 0)),
        out_shape=jax.ShapeDtypeStruct((2, N, 64), F32),
    )(h1, a2, W2, b2, W3)


def _l3_body(z3, a3, b3, Rm, h3_out, s_out):
    s = jnp.zeros((4, BN), F32)
    for p in range(2):
        ag = jnp.concatenate(
            [a3[0, 2 * p] + a3[1, 2 * p], a3[0, 2 * p + 1] + a3[1, 2 * p + 1]],
            axis=-1)
        hp = _gelu(z3[p] + ag + b3[0][None, p * 64:(p + 1) * 64])
        h3_out[p] = hp
        s = s + jax.lax.dot_general(Rm[p * 64:(p + 1) * 64, :], hp,
                                    (((0,), (1,)), ((), ())),
                                    preferred_element_type=F32)
    s_out[...] = s


def _run_l3(z3, a3, b3, Rm):
    return pl.pallas_call(
        _l3_body,
        grid=(N // BN,),
        in_specs=[
            pl.BlockSpec((2, BN, 64), lambda i: (0, i, 0)),
            pl.BlockSpec((2, 4, BN, 32), lambda i: (0, 0, i, 0)),
            pl.BlockSpec((1, 128), lambda i: (0, 0)),
            pl.BlockSpec((128, 4), lambda i: (0, 0)),
        ],
        out_specs=[
            pl.BlockSpec((2, BN, 64), lambda i: (0, i, 0)),
            pl.BlockSpec((4, BN), lambda i: (0, i)),
        ],
        out_shape=[
            jax.ShapeDtypeStruct((2, N, 64), F32),
            jax.ShapeDtypeStruct((4, N), F32),
        ],
    )(z3, a3, b3, Rm)


def _pre_body(cls, Wqkv, bqkv, Wout, bout, bert, Wbp, bbp, lbg, lbb,
              Rm_o, scls_o, btok_o, sbert_o, Mflat_o, c_o):
    q = jax.lax.dot_general(cls[...], Wqkv[:, 0:D], (((1,), (0,)), ((), ())),
                            preferred_element_type=F32) + bqkv[0][None, 0:D]
    Wk = Wqkv[:, D:2 * D]
    Wv = Wqkv[:, 2 * D:3 * D]
    cols = []
    for h in range(H):
        qh = q[:, h * DH:(h + 1) * DH]
        col = jax.lax.dot_general(Wk[:, h * DH:(h + 1) * DH], qh,
                                  (((1,), (1,)), ((), ())),
                                  preferred_element_type=F32)
        cols.append(col)
    Rm = jnp.concatenate(cols, axis=1) * (1.0 / np.sqrt(DH))
    Rm_o[...] = Rm
    scls_o[...] = jax.lax.dot_general(cls[...], Rm, (((1,), (0,)), ((), ())),
                                      preferred_element_type=F32)
    bt = jax.lax.dot_general(bert[...], Wbp[...], (((1,), (0,)), ((), ())),
                             preferred_element_type=F32) + bbp[0][None, :]
    mu = jnp.mean(bt, axis=-1, keepdims=True)
    var = jnp.mean((bt - mu) ** 2, axis=-1, keepdims=True)
    bt = (bt - mu) / jnp.sqrt(var + 1e-5) * lbg[0][None, :] + lbb[0][None, :]
    btok_o[...] = bt
    sbert_o[...] = jax.lax.dot_general(bt, Rm, (((1,), (0,)), ((), ())),
                                       preferred_element_type=F32)
    rows = []
    for h in range(H):
        rows.append(jax.lax.dot_general(Wv[:, h * DH:(h + 1) * DH],
                                        Wout[h * DH:(h + 1) * DH, :],
                                        (((1,), (0,)), ((), ())),
                                        preferred_element_type=F32))
    Mflat_o[...] = jnp.concatenate(rows, axis=0)
    c_o[...] = jax.lax.dot_general(bqkv[:, 2 * D:3 * D], Wout[...],
                                   (((1,), (0,)), ((), ())),
                                   preferred_element_type=F32) + bout[...]


def _run_pre(cls, Wqkv, bqkv, Wout, bout, bert, Wbp, bbp, lbg, lbb):
    full = lambda s: pl.BlockSpec(s, lambda: tuple(0 for _ in s))
    return pl.pallas_call(
        _pre_body,
        grid=(),
        in_specs=[full((1, 128)), full((128, 384)), full((1, 384)),
                  full((128, 128)), full((1, 128)), full((4, 1536)),
                  full((1536, 128)), full((1, 128)), full((1, 128)), full((1, 128))],
        out_specs=[full((128, 4)), full((1, 4)), full((4, 128)), full((4, 4)),
                   full((512, 128)), full((1, 128))],
        out_shape=[
            jax.ShapeDtypeStruct((128, 4), F32),
            jax.ShapeDtypeStruct((1, 4), F32),
            jax.ShapeDtypeStruct((4, 128), F32),
            jax.ShapeDtypeStruct((4, 4), F32),
            jax.ShapeDtypeStruct((512, 128), F32),
            jax.ShapeDtypeStruct((1, 128), F32),
        ],
    )(cls, Wqkv, bqkv, Wout, bout, bert, Wbp, bbp, lbg, lbb)


def _stats_body(s, scls, sbert, m_o, z_o):
    mx = jnp.max(s[...], axis=1)
    m = jnp.maximum(mx, jnp.maximum(scls[0], sbert[0, 0]))
    zs = jnp.sum(jnp.exp(s[...] - m[:, None]), axis=1)
    z = zs + jnp.exp(scls[0] - m) + jnp.exp(sbert[0, 0] - m)
    m_o[...] = m[None, None, :]
    z_o[...] = z[None, None, :]


def _run_stats(s, scls, sbert):
    return pl.pallas_call(
        _stats_body,
        grid=(B,),
        in_specs=[
            pl.BlockSpec((4, NPG), lambda b: (0, b)),
            pl.BlockSpec((1, 4), lambda b: (0, 0)),
            pl.BlockSpec((1, 1, 4), lambda b: (b, 0, 0)),
        ],
        out_specs=[pl.BlockSpec((1, 1, 4), lambda b: (b, 0, 0)),
                   pl.BlockSpec((1, 1, 4), lambda b: (b, 0, 0))],
        out_shape=[jax.ShapeDtypeStruct((B, 1, 4), F32),
                   jax.ShapeDtypeStruct((B, 1, 4), F32)],
    )(s, scls, sbert)


BN2 = 2048
NC2 = NPG // BN2


def _u_body(s, h3, m, z, scls, sbert, cls, btok, u_o):
    j = pl.program_id(1)
    a = jnp.exp(s[...] - m[0, 0][:, None]) / z[0, 0][:, None]
    parts = []
    for p in range(2):
        parts.append(jax.lax.dot_general(a, h3[p], (((1,), (0,)), ((), ())),
                                         preferred_element_type=F32))
    u = jnp.concatenate(parts, axis=1)

    @pl.when(j == 0)
    def _init():
        ec = jnp.exp(scls[0] - m[0, 0]) / z[0, 0]
        eb = jnp.exp(sbert[0, 0] - m[0, 0]) / z[0, 0]
        u_o[0] = (u + ec[:, None] * cls[0][None, :]
                  + eb[:, None] * btok[0, 0][None, :])

    @pl.when(j != 0)
    def _acc():
        u_o[0] = u_o[0] + u


def _run_u(s, h3, m, z, scls, sbert, cls, btok):
    return pl.pallas_call(
        _u_body,
        grid=(B, NC2),
        in_specs=[
            pl.BlockSpec((4, BN2), lambda b, j: (0, b * NC2 + j)),
            pl.BlockSpec((2, BN2, 64), lambda b, j: (0, b * NC2 + j, 0)),
            pl.BlockSpec((1, 1, 4), lambda b, j: (b, 0, 0)),
            pl.BlockSpec((1, 1, 4), lambda b, j: (b, 0, 0)),
            pl.BlockSpec((1, 4), lambda b, j: (0, 0)),
            pl.BlockSpec((1, 1, 4), lambda b, j: (b, 0, 0)),
            pl.BlockSpec((1, 128), lambda b, j: (0, 0)),
            pl.BlockSpec((1, 1, 128), lambda b, j: (b, 0, 0)),
        ],
        out_specs=pl.BlockSpec((1, 4, 128), lambda b, j: (b, 0, 0)),
        out_shape=jax.ShapeDtypeStruct((B, 4, 128), F32),
    )(s, h3, m, z, scls, sbert, cls, btok)


def _fin_body(u, Mflat, c, cls, lag, lab, out):
    x = c[0][None, :] + cls[...]
    acc = jnp.zeros((B, D), F32)
    for h in range(H):
        acc = acc + jax.lax.dot_general(u[:, h, :], Mflat[h * D:(h + 1) * D, :],
                                        (((1,), (0,)), ((), ())),
                                        preferred_element_type=F32)
    y = acc + x
    mu = jnp.mean(y, axis=-1, keepdims=True)
    var = jnp.mean((y - mu) ** 2, axis=-1, keepdims=True)
    out[...] = (y - mu) / jnp.sqrt(var + 1e-5) * lag[0][None, :] + lab[0][None, :]


def _run_fin(u, Mflat, c, cls, lag, lab):
    full = lambda s: pl.BlockSpec(s, lambda: tuple(0 for _ in s))
    return pl.pallas_call(
        _fin_body,
        grid=(),
        in_specs=[full((B, 4, 128)), full((512, 128)), full((1, 128)),
                  full((1, 128)), full((1, 128)), full((1, 128))],
        out_specs=full((B, 128)),
        out_shape=jax.ShapeDtypeStruct((B, 128), F32),
    )(u, Mflat, c, cls, lag, lab)


NW = 32
NPW = N // NW
EPT = E // NW
CH = 128
NCH = EPT // CH
RPT = N // 16


def _sc_mesh():
    from jax.experimental.pallas import tpu_sc as plsc
    return plsc.VectorSubcoreMesh(core_axis_name="c", subcore_axis_name="s")


def _embed_body(et_hbm, mask_hbm, h0_hbm, idx_v, idx2_v, rows_v, sem):
    from jax import lax
    wid = lax.axis_index("s") * 2 + lax.axis_index("c")
    base = wid * NPW
    pltpu.sync_copy(mask_hbm.at[pl.ds(base, NPW)], idx_v)
    pltpu.async_copy(et_hbm.at[idx_v], rows_v, sem).wait()
    pltpu.sync_copy(rows_v, h0_hbm.at[0, pl.ds(base, NPW)])

    def add_off(i, _):
        idx2_v[pl.ds(i * 16, 16)] = idx_v[pl.ds(i * 16, 16)] + 4096
        return 0
    jax.lax.fori_loop(0, NPW // 16, add_off, 0)
    pltpu.async_copy(et_hbm.at[idx2_v], rows_v, sem).wait()
    pltpu.sync_copy(rows_v, h0_hbm.at[1, pl.ds(base, NPW)])


def _embed_gather(embed_table, mask):
    etflat = embed_table.reshape(4096, 2, 64).transpose(1, 0, 2).reshape(2 * 4096, 64)
    run = pl.kernel(
        _embed_body,
        out_type=jax.ShapeDtypeStruct((2, N, 64), F32),
        mesh=_sc_mesh(),
        compiler_params=pltpu.CompilerParams(use_tc_tiling_on_sc=False, needs_layout_passes=False),
        scratch_types=[
            pltpu.VMEM((NPW,), jnp.int32),
            pltpu.VMEM((NPW,), jnp.int32),
            pltpu.VMEM((NPW, 64), F32),
            pltpu.SemaphoreType.DMA,
        ],
    )
    return run(etflat, mask)


def _make_edge_body(P):
    from jax.experimental.pallas import tpu_sc as plsc
    from jax import lax

    def body(yflat_hbm, src_hbm, dst_hbm, w_hbm, out_hbm,
             src_v, dst_v, w_v, idx_v, didx_v, gb, sb, zbuf, acc,
             gsem, ssem):
        cid = lax.axis_index("c")
        sid = lax.axis_index("s")
        wid = sid * 2 + cid
        ebase = wid * EPT
        pltpu.sync_copy(src_hbm.at[pl.ds(ebase, EPT)], src_v)
        pltpu.sync_copy(dst_hbm.at[pl.ds(ebase, EPT)], dst_v)
        pltpu.sync_copy(w_hbm.at[pl.ds(ebase, EPT)], w_v)

        def zb(i, _):
            for k in range(2):
                zbuf[i, pl.ds(k * 16, 16)] = jnp.zeros((16,), F32)
            return 0
        lax.fori_loop(0, 256, zb, 0)

        NB = 4
        NG = NCH // NB

        def build_idx(b, j, off):
            for k2 in range(CH // 16):
                idx_v[b, pl.ds(k2 * 16, 16)] = (
                    src_v[pl.ds(j * CH + k2 * 16, 16)] * 2 + off)

        def build_didx(b, j):
            for k2 in range(CH // 16):
                didx_v[b, pl.ds(k2 * 16, 16)] = dst_v[pl.ds(j * CH + k2 * 16, 16)]

        def g_issue(b):
            pltpu.async_copy(yflat_hbm.at[idx_v.at[b]], gb.at[b], gsem.at[b])

        def g_wait(b):
            pltpu.make_async_copy(yflat_hbm.at[idx_v.at[b]], gb.at[b],
                                  gsem.at[b]).wait()

        def s_issue(b):
            pltpu.async_copy(sb.at[b], acc.at[didx_v.at[b]], ssem.at[b],
                             add=True)

        def s_wait(b):
            pltpu.make_async_copy(sb.at[b], acc.at[didx_v.at[b]],
                                  ssem.at[b]).wait()

        for q in range(2 * P):
            off = (q // 2) * 2 * N + (q % 2)
            for k in range(4):
                pltpu.sync_copy(zbuf, acc.at[pl.ds(sid * RPT + k * 256, 256)])
            plsc.subcore_barrier()

            for b in range(NB):
                build_idx(b, b, off)
                g_issue(b)

            def group(jo, _):
                for b in range(NB):
                    j = jo * NB + b

                    @pl.when(jo > 0)
                    def _():
                        s_wait(b)
                    g_wait(b)

                    @plsc.parallel_loop(0, CH, step=1, unroll=4)
                    def scale(r):
                        wsp = plsc.load_gather(
                            w_v, [jnp.full((16,), j * CH + r, jnp.int32)])
                        for k in range(2):
                            sb[b, r, pl.ds(k * 16, 16)] = (
                                gb[b, r, pl.ds(k * 16, 16)] * wsp)
                    build_didx(b, j)
                    s_issue(b)

                    @pl.when(jo < NG - 1)
                    def _():
                        build_idx(b, j + NB, off)
                        g_issue(b)
                return 0
            lax.fori_loop(0, NG, group, 0)
            for b in range(NB):
                s_wait(b)
            plsc.subcore_barrier()
            pltpu.sync_copy(acc.at[pl.ds(sid * RPT, RPT)],
                            out_hbm.at[cid, q, pl.ds(sid * RPT, RPT)])
            plsc.subcore_barrier()
    return body


def _agg_chunks(y, src, dst, w, P):
    yflat = y.reshape(2 * P * N, 32)
    run = pl.kernel(
        _make_edge_body(P),
        out_type=jax.ShapeDtypeStruct((2, 2 * P, N, 32), F32),
        mesh=_sc_mesh(),
        compiler_params=pltpu.CompilerParams(use_tc_tiling_on_sc=False, needs_layout_passes=False),
        scratch_types=[
            pltpu.VMEM((EPT,), jnp.int32),
            pltpu.VMEM((EPT,), jnp.int32),
            pltpu.VMEM((EPT,), F32),
            pltpu.VMEM((4, CH), jnp.int32),
            pltpu.VMEM((4, CH), jnp.int32),
            pltpu.VMEM((4, CH, 32), F32),
            pltpu.VMEM((4, CH, 32), F32),
            pltpu.VMEM((256, 32), F32),
            pltpu.VMEM_SHARED((N, 32), F32),
            pltpu.SemaphoreType.DMA((4,)),
            pltpu.SemaphoreType.DMA((4,)),
        ],
    )
    return run(yflat, src, dst, w)


def kernel(mask, edge_index, sw, edge_weight, bert_feat, embed_table, W1, b1,
           W2, b2, W3, b3, cls_emb, Wbp, bbp, ln_b_g, ln_b_b, Wqkv, bqkv,
           Wout, bout, ln_a_g, ln_a_b):
    src = edge_index[0]
    dst = edge_index[1]
    row = lambda v: v.reshape(1, -1)

    cls = cls_emb.reshape(1, D)
    Rm, scls, btok, sbert, Mflat, c = _run_pre(
        cls, Wqkv, row(bqkv), Wout, row(bout), bert_feat, Wbp, row(bbp),
        row(ln_b_g), row(ln_b_b))

    h0 = _embed_gather(embed_table, mask)
    a1 = _agg_chunks(h0, src, dst, edge_weight, 2)
    h1 = _run_l1(h0, a1, W1, row(b1))
    a2 = _agg_chunks(h1, src, dst, edge_weight, 4)
    z3 = _run_l2(h1, a2, W2, row(b2), W3)
    a3 = _agg_chunks(z3, src, dst, edge_weight, 2)
    h3, s = _run_l3(z3, a3, row(b3), Rm)

    sbert_r = sbert.reshape(B, 1, 4)
    btok_r = btok.reshape(B, 1, 128)
    m, z = _run_stats(s, scls, sbert_r)
    u = _run_u(s, h3, m, z, scls, sbert_r, cls, btok_r)
    return _run_fin(u, Mflat, c, cls, row(ln_a_g), row(ln_a_b))


# speedup vs baseline: 2.7475x; 1.0261x over previous
import functools

import jax
import jax.numpy as jnp
import numpy as np
from jax.experimental import pallas as pl
from jax.experimental.pallas import tpu as pltpu

NPG = 4096
D = 128
H = 4
DH = D // H
B = 4
N = B * NPG
E = N * 32
BN = 1024
F32 = jnp.float32


def _gelu(x):
    return 0.5 * x * (1.0 + jax.lax.erf(x * np.float32(1.0 / np.sqrt(2.0))))


def _l1_body(h0, a1, W1, b1, out):
    z = jnp.zeros((BN, 256), F32)
    for p in range(2):
        z = z + jax.lax.dot_general(h0[p], W1[p * 64:(p + 1) * 64, :],
                                    (((1,), (0,)), ((), ())),
                                    preferred_element_type=F32)
    for q in range(4):
        x = a1[0, q] + a1[1, q]
        z = z + jax.lax.dot_general(x, W1[q * 32:(q + 1) * 32, :],
                                    (((1,), (0,)), ((), ())),
                                    preferred_element_type=F32)
    hv = _gelu(z + b1[0][None, :])
    for p in range(4):
        out[p] = hv[:, p * 64:(p + 1) * 64]


def _run_l1(h0, a1, W1, b1):
    return pl.pallas_call(
        _l1_body,
        grid=(N // BN,),
        in_specs=[
            pl.BlockSpec((2, BN, 64), lambda i: (0, i, 0)),
            pl.BlockSpec((2, 4, BN, 32), lambda i: (0, 0, i, 0)),
            pl.BlockSpec((128, 256), lambda i: (0, 0)),
            pl.BlockSpec((1, 256), lambda i: (0, 0)),
        ],
        out_specs=pl.BlockSpec((4, BN, 64), lambda i: (0, i, 0)),
        out_shape=jax.ShapeDtypeStruct((4, N, 64), F32),
    )(h0, a1, W1, b1)


def _l2_body(h1, a2, W2, b2, W3, out):
    z = jnp.zeros((BN, 256), F32)
    for p in range(4):
        z = z + jax.lax.dot_general(h1[p], W2[p * 64:(p + 1) * 64, :],
                                    (((1,), (0,)), ((), ())),
                                    preferred_element_type=F32)
    for q in range(8):
        x = a2[0, q] + a2[1, q]
        z = z + jax.lax.dot_general(x, W2[q * 32:(q + 1) * 32, :],
                                    (((1,), (0,)), ((), ())),
                                    preferred_element_type=F32)
    h2 = _gelu(z + b2[0][None, :])
    z3 = jax.lax.dot_general(h2, W3[...], (((1,), (0,)), ((), ())),
                             preferred_element_type=F32)
    for p in range(2):
        out[p] = z3[:, p * 64:(p + 1) * 64]


def _run_l2(h1, a2, W2, b2, W3):
    return pl.pallas_call(
        _l2_body,
        grid=(N // BN,),
        in_specs=[
            pl.BlockSpec((4, BN, 64), lambda i: (0, i, 0)),
            pl.BlockSpec((2, 8, BN, 32), lambda i: (0, 0, i, 0)),
            pl.BlockSpec((256, 256), lambda i: (0, 0)),
            pl.BlockSpec((1, 256), lambda i: (0, 0)),
            pl.BlockSpec((256, 128), lambda i: (0, 0)),
        ],
        out_specs=pl.BlockSpec((2, BN, 64), lambda i: (0, i, 0)),
        out_shape=jax.ShapeDtypeStruct((2, N, 64), F32),
    )(h1, a2, W2, b2, W3)


def _l3_body(z3, a3, b3, Rm, h3_out, s_out):
    s = jnp.zeros((4, BN), F32)
    for p in range(2):
        ag = jnp.concatenate(
            [a3[0, 2 * p] + a3[1, 2 * p], a3[0, 2 * p + 1] + a3[1, 2 * p + 1]],
            axis=-1)
        hp = _gelu(z3[p] + ag + b3[0][None, p * 64:(p + 1) * 64])
        h3_out[p] = hp
        s = s + jax.lax.dot_general(Rm[p * 64:(p + 1) * 64, :], hp,
                                    (((0,), (1,)), ((), ())),
                                    preferred_element_type=F32)
    s_out[...] = s


def _run_l3(z3, a3, b3, Rm):
    return pl.pallas_call(
        _l3_body,
        grid=(N // BN,),
        in_specs=[
            pl.BlockSpec((2, BN, 64), lambda i: (0, i, 0)),
            pl.BlockSpec((2, 4, BN, 32), lambda i: (0, 0, i, 0)),
            pl.BlockSpec((1, 128), lambda i: (0, 0)),
            pl.BlockSpec((128, 4), lambda i: (0, 0)),
        ],
        out_specs=[
            pl.BlockSpec((2, BN, 64), lambda i: (0, i, 0)),
            pl.BlockSpec((4, BN), lambda i: (0, i)),
        ],
        out_shape=[
            jax.ShapeDtypeStruct((2, N, 64), F32),
            jax.ShapeDtypeStruct((4, N), F32),
        ],
    )(z3, a3, b3, Rm)


def _pre_body(cls, Wqkv, bqkv, Wout, bout, bert, Wbp, bbp, lbg, lbb,
              Rm_o, scls_o, btok_o, sbert_o, Mflat_o, c_o):
    q = jax.lax.dot_general(cls[...], Wqkv[:, 0:D], (((1,), (0,)), ((), ())),
                            preferred_element_type=F32) + bqkv[0][None, 0:D]
    Wk = Wqkv[:, D:2 * D]
    Wv = Wqkv[:, 2 * D:3 * D]
    cols = []
    for h in range(H):
        qh = q[:, h * DH:(h + 1) * DH]
        col = jax.lax.dot_general(Wk[:, h * DH:(h + 1) * DH], qh,
                                  (((1,), (1,)), ((), ())),
                                  preferred_element_type=F32)
        cols.append(col)
    Rm = jnp.concatenate(cols, axis=1) * (1.0 / np.sqrt(DH))
    Rm_o[...] = Rm
    scls_o[...] = jax.lax.dot_general(cls[...], Rm, (((1,), (0,)), ((), ())),
                                      preferred_element_type=F32)
    bt = jax.lax.dot_general(bert[...], Wbp[...], (((1,), (0,)), ((), ())),
                             preferred_element_type=F32) + bbp[0][None, :]
    mu = jnp.mean(bt, axis=-1, keepdims=True)
    var = jnp.mean((bt - mu) ** 2, axis=-1, keepdims=True)
    bt = (bt - mu) / jnp.sqrt(var + 1e-5) * lbg[0][None, :] + lbb[0][None, :]
    btok_o[...] = bt
    sbert_o[...] = jax.lax.dot_general(bt, Rm, (((1,), (0,)), ((), ())),
                                       preferred_element_type=F32)
    rows = []
    for h in range(H):
        rows.append(jax.lax.dot_general(Wv[:, h * DH:(h + 1) * DH],
                                        Wout[h * DH:(h + 1) * DH, :],
                                        (((1,), (0,)), ((), ())),
                                        preferred_element_type=F32))
    Mflat_o[...] = jnp.concatenate(rows, axis=0)
    c_o[...] = jax.lax.dot_general(bqkv[:, 2 * D:3 * D], Wout[...],
                                   (((1,), (0,)), ((), ())),
                                   preferred_element_type=F32) + bout[...]


def _run_pre(cls, Wqkv, bqkv, Wout, bout, bert, Wbp, bbp, lbg, lbb):
    full = lambda s: pl.BlockSpec(s, lambda: tuple(0 for _ in s))
    return pl.pallas_call(
        _pre_body,
        grid=(),
        in_specs=[full((1, 128)), full((128, 384)), full((1, 384)),
                  full((128, 128)), full((1, 128)), full((4, 1536)),
                  full((1536, 128)), full((1, 128)), full((1, 128)), full((1, 128))],
        out_specs=[full((128, 4)), full((1, 4)), full((4, 128)), full((4, 4)),
                   full((512, 128)), full((1, 128))],
        out_shape=[
            jax.ShapeDtypeStruct((128, 4), F32),
            jax.ShapeDtypeStruct((1, 4), F32),
            jax.ShapeDtypeStruct((4, 128), F32),
            jax.ShapeDtypeStruct((4, 4), F32),
            jax.ShapeDtypeStruct((512, 128), F32),
            jax.ShapeDtypeStruct((1, 128), F32),
        ],
    )(cls, Wqkv, bqkv, Wout, bout, bert, Wbp, bbp, lbg, lbb)


def _tail_body(s, h3, scls, sbert, cls, btok, Mflat, c, lag, lab, out):
    sblk = s[...]
    m = jnp.maximum(jnp.max(sblk, axis=1),
                    jnp.maximum(scls[0], sbert[0, 0]))
    e = jnp.exp(sblk - m[:, None])
    ec = jnp.exp(scls[0] - m)
    eb = jnp.exp(sbert[0, 0] - m)
    z = jnp.sum(e, axis=1) + ec + eb
    a = e / z[:, None]
    parts = []
    for p in range(2):
        parts.append(jax.lax.dot_general(a, h3[p], (((1,), (0,)), ((), ())),
                                         preferred_element_type=F32))
    u = jnp.concatenate(parts, axis=1)
    u = (u + (ec / z)[:, None] * cls[0][None, :]
         + (eb / z)[:, None] * btok[0, 0][None, :])
    attn = jnp.zeros((1, D), F32)
    for h in range(H):
        attn = attn + jax.lax.dot_general(u[h:h + 1, :],
                                          Mflat[h * D:(h + 1) * D, :],
                                          (((1,), (0,)), ((), ())),
                                          preferred_element_type=F32)
    y = cls[...] + attn + c[...]
    mu = jnp.mean(y, axis=-1, keepdims=True)
    var = jnp.mean((y - mu) ** 2, axis=-1, keepdims=True)
    out[0] = (y - mu) / jnp.sqrt(var + 1e-5) * lag[...] + lab[...]


def _run_tail(s, h3, scls, sbert, cls, btok, Mflat, c, lag, lab):
    return pl.pallas_call(
        _tail_body,
        grid=(B,),
        in_specs=[
            pl.BlockSpec((4, NPG), lambda b: (0, b)),
            pl.BlockSpec((2, NPG, 64), lambda b: (0, b, 0)),
            pl.BlockSpec((1, 4), lambda b: (0, 0)),
            pl.BlockSpec((1, 1, 4), lambda b: (b, 0, 0)),
            pl.BlockSpec((1, 128), lambda b: (0, 0)),
            pl.BlockSpec((1, 1, 128), lambda b: (b, 0, 0)),
            pl.BlockSpec((512, 128), lambda b: (0, 0)),
            pl.BlockSpec((1, 128), lambda b: (0, 0)),
            pl.BlockSpec((1, 128), lambda b: (0, 0)),
            pl.BlockSpec((1, 128), lambda b: (0, 0)),
        ],
        out_specs=pl.BlockSpec((1, 1, 128), lambda b: (b, 0, 0)),
        out_shape=jax.ShapeDtypeStruct((B, 1, 128), F32),
    )(s, h3, scls, sbert, cls, btok, Mflat, c, lag, lab)


NW = 32
NPW = N // NW
EPT = E // NW
CH = 128
NCH = EPT // CH
RPT = N // 16


def _sc_mesh():
    from jax.experimental.pallas import tpu_sc as plsc
    return plsc.VectorSubcoreMesh(core_axis_name="c", subcore_axis_name="s")


def _embed_body(et_hbm, mask_hbm, h0_hbm, idx_v, idx2_v, rows_v, sem):
    from jax import lax
    wid = lax.axis_index("s") * 2 + lax.axis_index("c")
    base = wid * NPW
    pltpu.sync_copy(mask_hbm.at[pl.ds(base, NPW)], idx_v)
    pltpu.async_copy(et_hbm.at[idx_v], rows_v, sem).wait()
    pltpu.sync_copy(rows_v, h0_hbm.at[0, pl.ds(base, NPW)])

    def add_off(i, _):
        idx2_v[pl.ds(i * 16, 16)] = idx_v[pl.ds(i * 16, 16)] + 4096
        return 0
    jax.lax.fori_loop(0, NPW // 16, add_off, 0)
    pltpu.async_copy(et_hbm.at[idx2_v], rows_v, sem).wait()
    pltpu.sync_copy(rows_v, h0_hbm.at[1, pl.ds(base, NPW)])


def _embed_gather(embed_table, mask):
    etflat = embed_table.reshape(4096, 2, 64).transpose(1, 0, 2).reshape(2 * 4096, 64)
    run = pl.kernel(
        _embed_body,
        out_type=jax.ShapeDtypeStruct((2, N, 64), F32),
        mesh=_sc_mesh(),
        compiler_params=pltpu.CompilerParams(use_tc_tiling_on_sc=False, needs_layout_passes=False),
        scratch_types=[
            pltpu.VMEM((NPW,), jnp.int32),
            pltpu.VMEM((NPW,), jnp.int32),
            pltpu.VMEM((NPW, 64), F32),
            pltpu.SemaphoreType.DMA,
        ],
    )
    return run(etflat, mask)


def _make_edge_body(P):
    from jax.experimental.pallas import tpu_sc as plsc
    from jax import lax

    def body(yflat_hbm, src_hbm, dst_hbm, w_hbm, out_hbm,
             src_v, dst_v, w_v, idx_v, didx_v, gb, sb, zbuf, acc,
             gsem, ssem):
        cid = lax.axis_index("c")
        sid = lax.axis_index("s")
        wid = sid * 2 + cid
        ebase = wid * EPT
        pltpu.sync_copy(src_hbm.at[pl.ds(ebase, EPT)], src_v)
        pltpu.sync_copy(dst_hbm.at[pl.ds(ebase, EPT)], dst_v)
        pltpu.sync_copy(w_hbm.at[pl.ds(ebase, EPT)], w_v)

        def zb(i, _):
            for k in range(2):
                zbuf[i, pl.ds(k * 16, 16)] = jnp.zeros((16,), F32)
            return 0
        lax.fori_loop(0, 256, zb, 0)

        NB = 4
        NG = NCH // NB

        def build_idx(b, j, off):
            for k2 in range(CH // 16):
                idx_v[b, pl.ds(k2 * 16, 16)] = (
                    src_v[pl.ds(j * CH + k2 * 16, 16)] * 2 + off)

        def build_didx(b, j):
            for k2 in range(CH // 16):
                didx_v[b, pl.ds(k2 * 16, 16)] = dst_v[pl.ds(j * CH + k2 * 16, 16)]

        def g_issue(b):
            pltpu.async_copy(yflat_hbm.at[idx_v.at[b]], gb.at[b], gsem.at[b])

        def g_wait(b):
            pltpu.make_async_copy(yflat_hbm.at[idx_v.at[b]], gb.at[b],
                                  gsem.at[b]).wait()

        def s_issue(b):
            pltpu.async_copy(sb.at[b], acc.at[didx_v.at[b]], ssem.at[b],
                             add=True)

        def s_wait(b):
            pltpu.make_async_copy(sb.at[b], acc.at[didx_v.at[b]],
                                  ssem.at[b]).wait()

        for q in range(2 * P):
            off = (q // 2) * 2 * N + (q % 2)
            for k in range(4):
                pltpu.sync_copy(zbuf, acc.at[pl.ds(sid * RPT + k * 256, 256)])
            plsc.subcore_barrier()

            for b in range(NB):
                build_idx(b, b, off)
                g_issue(b)

            def group(jo, _):
                for b in range(NB):
                    j = jo * NB + b

                    @pl.when(jo > 0)
                    def _():
                        s_wait(b)
                    g_wait(b)

                    @plsc.parallel_loop(0, CH, step=1, unroll=4)
                    def scale(r):
                        wsp = plsc.load_gather(
                            w_v, [jnp.full((16,), j * CH + r, jnp.int32)])
                        for k in range(2):
                            sb[b, r, pl.ds(k * 16, 16)] = (
                                gb[b, r, pl.ds(k * 16, 16)] * wsp)
                    build_didx(b, j)
                    s_issue(b)

                    @pl.when(jo < NG - 1)
                    def _():
                        build_idx(b, j + NB, off)
                        g_issue(b)
                return 0
            lax.fori_loop(0, NG, group, 0)
            for b in range(NB):
                s_wait(b)
            plsc.subcore_barrier()
            pltpu.sync_copy(acc.at[pl.ds(sid * RPT, RPT)],
                            out_hbm.at[cid, q, pl.ds(sid * RPT, RPT)])
            plsc.subcore_barrier()
    return body


def _agg_chunks(y, src, dst, w, P):
    yflat = y.reshape(2 * P * N, 32)
    run = pl.kernel(
        _make_edge_body(P),
        out_type=jax.ShapeDtypeStruct((2, 2 * P, N, 32), F32),
        mesh=_sc_mesh(),
        compiler_params=pltpu.CompilerParams(use_tc_tiling_on_sc=False, needs_layout_passes=False),
        scratch_types=[
            pltpu.VMEM((EPT,), jnp.int32),
            pltpu.VMEM((EPT,), jnp.int32),
            pltpu.VMEM((EPT,), F32),
            pltpu.VMEM((4, CH), jnp.int32),
            pltpu.VMEM((4, CH), jnp.int32),
            pltpu.VMEM((4, CH, 32), F32),
            pltpu.VMEM((4, CH, 32), F32),
            pltpu.VMEM((256, 32), F32),
            pltpu.VMEM_SHARED((N, 32), F32),
            pltpu.SemaphoreType.DMA((4,)),
            pltpu.SemaphoreType.DMA((4,)),
        ],
    )
    return run(yflat, src, dst, w)


def kernel(mask, edge_index, sw, edge_weight, bert_feat, embed_table, W1, b1,
           W2, b2, W3, b3, cls_emb, Wbp, bbp, ln_b_g, ln_b_b, Wqkv, bqkv,
           Wout, bout, ln_a_g, ln_a_b):
    src = edge_index[0]
    dst = edge_index[1]
    row = lambda v: v.reshape(1, -1)

    cls = cls_emb.reshape(1, D)
    Rm, scls, btok, sbert, Mflat, c = _run_pre(
        cls, Wqkv, row(bqkv), Wout, row(bout), bert_feat, Wbp, row(bbp),
        row(ln_b_g), row(ln_b_b))

    h0 = _embed_gather(embed_table, mask)
    a1 = _agg_chunks(h0, src, dst, edge_weight, 2)
    h1 = _run_l1(h0, a1, W1, row(b1))
    a2 = _agg_chunks(h1, src, dst, edge_weight, 4)
    z3 = _run_l2(h1, a2, W2, row(b2), W3)
    a3 = _agg_chunks(z3, src, dst, edge_weight, 2)
    h3, s = _run_l3(z3, a3, row(b3), Rm)

    sbert_r = sbert.reshape(B, 1, 4)
    btok_r = btok.reshape(B, 1, 128)
    out = _run_tail(s, h3, scls, sbert_r, cls, btok_r, Mflat, c,
                    row(ln_a_g), row(ln_a_b))
    return out.reshape(B, D)
